# baseline (speedup 1.0000x reference)
import functools

import jax
import jax.numpy as jnp
import numpy as np
from jax import lax
from jax.experimental import pallas as pl
from jax.experimental.pallas import tpu as pltpu

_R = 32
_C = 1000000
_BLK = 4096
_NB = (_C + _BLK - 1) // _BLK

_K0 = np.int32(0)
_K1 = np.int32(42)
_K2 = np.int32(np.uint32(0) ^ np.uint32(42) ^ np.uint32(0x1BD11BDA))
_ROT0 = (13, 15, 26, 6)
_ROT1 = (17, 29, 16, 24)
_TINY = np.float32(np.finfo(np.float32).tiny)
_NEG_INF = np.float32(-np.inf)
_IMAX = np.int32(2**31 - 1)


def _rotl(v, r):
    return lax.shift_left(v, np.int32(r)) | lax.shift_right_logical(
        v, np.int32(32 - r))


def _rounds(x0, x1, rots):
    for r in rots:
        x0 = x0 + x1
        x1 = _rotl(x1, r)
        x1 = x0 ^ x1
    return x0, x1


def _threefry_bits(idx):
    x0 = jnp.full_like(idx, _K0)
    x1 = idx + _K1
    x0, x1 = _rounds(x0, x1, _ROT0)
    x0 = x0 + _K1
    x1 = x1 + (_K2 + np.int32(1))
    x0, x1 = _rounds(x0, x1, _ROT1)
    x0 = x0 + _K2
    x1 = x1 + (_K0 + np.int32(2))
    x0, x1 = _rounds(x0, x1, _ROT0)
    x0 = x0 + _K0
    x1 = x1 + (_K1 + np.int32(3))
    x0, x1 = _rounds(x0, x1, _ROT1)
    x0 = x0 + _K1
    x1 = x1 + (_K2 + np.int32(4))
    x0, x1 = _rounds(x0, x1, _ROT0)
    x0 = x0 + _K2
    x1 = x1 + (_K0 + np.int32(5))
    return x0 ^ x1


def _sample_kernel(x_ref, o_ref, best_v, best_i):
    p = pl.program_id(0)

    @pl.when(p == 0)
    def _init():
        best_v[...] = jnp.full((_R, 1), _NEG_INF, jnp.float32)
        best_i[...] = jnp.zeros((_R, 1), jnp.int32)

    col = lax.broadcasted_iota(jnp.int32, (_R, _BLK), 1) + p * np.int32(_BLK)
    row = lax.broadcasted_iota(jnp.int32, (_R, _BLK), 0)
    idx = row * np.int32(_C) + col

    bits = _threefry_bits(idx)
    fb = lax.shift_right_logical(bits, np.int32(9)) | np.int32(0x3F800000)
    f = lax.bitcast_convert_type(fb, jnp.float32) - np.float32(1.0)
    u = jnp.maximum(_TINY, f * (np.float32(1.0) - _TINY) + _TINY)
    g = -jnp.log(-jnp.log(u))
    score = g + jnp.log(x_ref[...])
    score = jnp.where(col < np.int32(_C), score, _NEG_INF)

    m = jnp.max(score, axis=1, keepdims=True)
    arg = jnp.min(jnp.where(score == m, col, _IMAX), axis=1, keepdims=True)

    upd = m > best_v[...]
    best_v[...] = jnp.where(upd, m, best_v[...])
    best_i[...] = jnp.where(upd, arg, best_i[...])

    @pl.when(p == _NB - 1)
    def _fin():
        o_ref[...] = best_i[...]


@jax.jit
def kernel(x):
    return pl.pallas_call(
        _sample_kernel,
        grid=(_NB,),
        in_specs=[pl.BlockSpec((_R, _BLK), lambda p: (0, p))],
        out_specs=pl.BlockSpec((_R, 1), lambda p: (0, 0)),
        out_shape=jax.ShapeDtypeStruct((_R, 1), jnp.int32),
        scratch_shapes=[
            pltpu.VMEM((_R, 1), jnp.float32),
            pltpu.VMEM((_R, 1), jnp.int32),
        ],
    )(x)


# speedup vs baseline: 1.2844x; 1.2844x over previous
import jax
import jax.numpy as jnp
import numpy as np
from jax import lax
from jax.experimental import pallas as pl
from jax.experimental.pallas import tpu as pltpu

_R = 32
_C = 1000000
_BLK = 4096
_W = 256
_NCH = _BLK // _W
_NBF = _C // _BLK
_TAIL_START = _NBF * _BLK
_TAIL_BLK = 1024
_TAIL_NCH = _TAIL_BLK // _W

_K1 = np.int32(42)
_K2 = np.int32(np.uint32(42) ^ np.uint32(0x1BD11BDA))
_ROT0 = (13, 15, 26, 6)
_ROT1 = (17, 29, 16, 24)
_TINY = np.float32(np.finfo(np.float32).tiny)
_NEG_INF = np.float32(-np.inf)
_IMAX = np.int32(2**31 - 1)


def _rotl(v, r):
    return lax.shift_left(v, np.int32(r)) | lax.shift_right_logical(
        v, np.int32(32 - r))


def _rounds(x0, x1, rots):
    for r in rots:
        x0 = x0 + x1
        x1 = _rotl(x1, r)
        x1 = x0 ^ x1
    return x0, x1


def _threefry_bits(x1):
    x0 = x1
    t = _rotl(x1, 13)
    x1 = x0 ^ t
    for r in _ROT0[1:]:
        x0 = x0 + x1
        x1 = _rotl(x1, r)
        x1 = x0 ^ x1
    x0 = x0 + _K1
    x1 = x1 + np.int32(_K2 + np.uint32(1))
    x0, x1 = _rounds(x0, x1, _ROT1)
    x0 = x0 + _K2
    x1 = x1 + np.int32(2)
    x0, x1 = _rounds(x0, x1, _ROT0)
    x0 = x0
    x1 = x1 + np.int32(_K1 + np.uint32(3))
    x0, x1 = _rounds(x0, x1, _ROT1)
    x0 = x0 + _K1
    x1 = x1 + np.int32(_K2 + np.uint32(4))
    x0, x1 = _rounds(x0, x1, _ROT0)
    x0 = x0 + _K2
    x1 = x1 + np.int32(5)
    return x0 ^ x1


def _score(xv, x1_0):
    bits = _threefry_bits(x1_0)
    fb = lax.shift_right_logical(bits, np.int32(9)) | np.int32(0x3F800000)
    f = lax.bitcast_convert_type(fb, jnp.float32) - np.float32(1.0)
    u = f + _TINY
    return -jnp.log(-jnp.log(u)) + jnp.log(xv)


def _sample_kernel(x_ref, xt_ref, o_ref, bv_ref, bc_ref):
    p = pl.program_id(0)
    row = lax.broadcasted_iota(jnp.int32, (_R, _W), 0)
    lane = lax.broadcasted_iota(jnp.int32, (_R, _W), 1)
    base = row * np.int32(_C) + lane + _K1

    def chunk(col0, xv):
        x1_0 = base + np.int32(col0)
        return _score(xv, x1_0), x1_0

    bv = None
    start = p * np.int32(_BLK)
    for ch in range(_NCH):
        xv = x_ref[:, ch * _W:(ch + 1) * _W]
        c = base + start + np.int32(ch * _W)
        s = _score(xv, c)
        if bv is None:
            bv, bc = s, c
        else:
            upd = s > bv
            bv = jnp.where(upd, s, bv)
            bc = jnp.where(upd, c, bc)

    @pl.when(p == 0)
    def _init():
        bv_ref[...] = bv
        bc_ref[...] = bc

    @pl.when(p > 0)
    def _merge():
        ov = bv_ref[...]
        upd = bv > ov
        bv_ref[...] = jnp.where(upd, bv, ov)
        bc_ref[...] = jnp.where(upd, bc, bc_ref[...])

    @pl.when(p == _NBF - 1)
    def _fin():
        mv = bv_ref[...]
        mc = bc_ref[...]
        for tc in range(_TAIL_NCH):
            col0 = _TAIL_START + tc * _W
            xv = xt_ref[:, tc * _W:(tc + 1) * _W]
            s, c = chunk(col0, xv)
            col = lane + np.int32(col0)
            s = jnp.where(col < np.int32(_C), s, _NEG_INF)
            upd = s > mv
            mv = jnp.where(upd, s, mv)
            mc = jnp.where(upd, c, mc)
        m = jnp.max(mv, axis=1, keepdims=True)
        arg = jnp.min(jnp.where(mv == m, mc, _IMAX), axis=1, keepdims=True)
        rowc = lax.broadcasted_iota(jnp.int32, (_R, 1), 0) * np.int32(_C)
        o_ref[...] = arg - rowc - _K1


@jax.jit
def kernel(x):
    return pl.pallas_call(
        _sample_kernel,
        grid=(_NBF,),
        in_specs=[
            pl.BlockSpec((_R, _BLK), lambda p: (0, p)),
            pl.BlockSpec((_R, _TAIL_BLK),
                         lambda p: (0, _TAIL_START // _TAIL_BLK)),
        ],
        out_specs=pl.BlockSpec((_R, 1), lambda p: (0, 0)),
        out_shape=jax.ShapeDtypeStruct((_R, 1), jnp.int32),
        scratch_shapes=[
            pltpu.VMEM((_R, _W), jnp.float32),
            pltpu.VMEM((_R, _W), jnp.int32),
        ],
    )(x, x)


# speedup vs baseline: 1.2956x; 1.0087x over previous
import jax
import jax.numpy as jnp
import numpy as np
from jax import lax
from jax.experimental import pallas as pl
from jax.experimental.pallas import tpu as pltpu

_R = 32
_C = 1000000
_BLK = 8192
_W = 128
_NCH = _BLK // _W
_NBF = _C // _BLK
_TAIL_START = _NBF * _BLK
_TAIL_BLK = 1024
_TAIL_NCH = _TAIL_BLK // _W

_K1 = np.int32(42)
_K2 = np.int32(np.uint32(42) ^ np.uint32(0x1BD11BDA))
_ROT0 = (13, 15, 26, 6)
_ROT1 = (17, 29, 16, 24)
_TINY = np.float32(np.finfo(np.float32).tiny)
_NEG_INF = np.float32(-np.inf)
_IMAX = np.int32(2**31 - 1)


def _rotl(v, r):
    return lax.shift_left(v, np.int32(r)) | lax.shift_right_logical(
        v, np.int32(32 - r))


def _rounds(x0, x1, rots):
    for r in rots:
        x0 = x0 + x1
        x1 = _rotl(x1, r)
        x1 = x0 ^ x1
    return x0, x1


def _threefry_bits(x1):
    x0 = x1
    x1 = x0 ^ _rotl(x1, 13)
    for r in _ROT0[1:]:
        x0 = x0 + x1
        x1 = _rotl(x1, r)
        x1 = x0 ^ x1
    x0 = x0 + _K1
    x1 = x1 + np.int32(_K2 + np.uint32(1))
    x0, x1 = _rounds(x0, x1, _ROT1)
    x0 = x0 + _K2
    x1 = x1 + np.int32(2)
    x0, x1 = _rounds(x0, x1, _ROT0)
    x1 = x1 + np.int32(_K1 + np.uint32(3))
    x0, x1 = _rounds(x0, x1, _ROT1)
    x0 = x0 + _K1
    x1 = x1 + np.int32(_K2 + np.uint32(4))
    x0, x1 = _rounds(x0, x1, _ROT0)
    x0 = x0 + _K2
    x1 = x1 + np.int32(5)
    return x0 ^ x1


def _score(xv, x1_0):
    bits = _threefry_bits(x1_0)
    fb = lax.shift_right_logical(bits, np.int32(9)) | np.int32(0x3F800000)
    f = lax.bitcast_convert_type(fb, jnp.float32) - np.float32(1.0)
    u = f + _TINY
    return -jnp.log(-jnp.log(u)) + jnp.log(xv)


def _run_chunks(x_ref, basec, start, chunks):
    bv = bc = None
    for ch in chunks:
        xv = x_ref[:, ch * _W:(ch + 1) * _W]
        c = basec + (start + np.int32(ch * _W))
        s = _score(xv, c)
        if bv is None:
            bv, bc = s, c
        else:
            upd = s > bv
            bv = jnp.where(upd, s, bv)
            bc = jnp.where(upd, c, bc)
    return bv, bc


def _sample_kernel(base_ref, x_ref, xt_ref, o_ref, bv_ref, bc_ref):
    p = pl.program_id(0)
    basec = base_ref[...]

    start = p * np.int32(_BLK)
    h = _NCH // 2
    bva, bca = _run_chunks(x_ref, basec, start, range(h))
    bvb, bcb = _run_chunks(x_ref, basec, start, range(h, _NCH))
    updh = bvb > bva
    bv = jnp.where(updh, bvb, bva)
    bc = jnp.where(updh, bcb, bca)

    @pl.when(p == 0)
    def _init():
        bv_ref[...] = bv
        bc_ref[...] = bc

    @pl.when(p > 0)
    def _merge():
        ov = bv_ref[...]
        upd = bv > ov
        bv_ref[...] = jnp.where(upd, bv, ov)
        bc_ref[...] = jnp.where(upd, bc, bc_ref[...])

    @pl.when(p == _NBF - 1)
    def _fin():
        mv = bv_ref[...]
        mc = bc_ref[...]
        lane = basec - basec[:, :1]
        for tc in range(_TAIL_NCH):
            col0 = _TAIL_START + tc * _W
            xv = xt_ref[:, tc * _W:(tc + 1) * _W]
            c = basec + np.int32(col0)
            s = _score(xv, c)
            s = jnp.where(lane < np.int32(_C - col0), s, _NEG_INF)
            upd = s > mv
            mv = jnp.where(upd, s, mv)
            mc = jnp.where(upd, c, mc)
        m = jnp.max(mv, axis=1, keepdims=True)
        arg = jnp.min(jnp.where(mv == m, mc, _IMAX), axis=1, keepdims=True)
        o_ref[...] = arg - basec[:, :1]


@jax.jit
def kernel(x):
    base = (np.arange(_R, dtype=np.int32)[:, None] * _C +
            np.arange(_W, dtype=np.int32)[None, :] + 42)
    return pl.pallas_call(
        _sample_kernel,
        grid=(_NBF,),
        in_specs=[
            pl.BlockSpec((_R, _W), lambda p: (0, 0)),
            pl.BlockSpec((_R, _BLK), lambda p: (0, p)),
            pl.BlockSpec((_R, _TAIL_BLK),
                         lambda p: (0, _TAIL_START // _TAIL_BLK)),
        ],
        out_specs=pl.BlockSpec((_R, 1), lambda p: (0, 0)),
        out_shape=jax.ShapeDtypeStruct((_R, 1), jnp.int32),
        scratch_shapes=[
            pltpu.VMEM((_R, _W), jnp.float32),
            pltpu.VMEM((_R, _W), jnp.int32),
        ],
    )(jnp.asarray(base), x, x)
